# baseline (speedup 1.0000x reference)
import functools

import jax
import jax.numpy as jnp
from jax import lax
from jax.experimental import pallas as pl
from jax.experimental.pallas import tpu as pltpu
from jax.experimental.pallas import tpu_sc as plsc

VOCAB = 100000
EMBED = 64
NFEAT = 26
HIDDEN = 1024
NCLASS = 2
BATCH = 4096

_NC = 2
_NS = 16
_NW = _NC * _NS

_ROWS = BATCH * NFEAT
_BPW = BATCH // _NW
_BPC = 16
_NCHUNK = _BPW // _BPC
_NF32 = 32
_CROWS = _BPC * _NF32


def _gather_sc(x2d, emb):
    mesh = plsc.VectorSubcoreMesh(core_axis_name="c", subcore_axis_name="s")

    @functools.partial(
        pl.kernel,
        mesh=mesh,
        compiler_params=pltpu.CompilerParams(use_tc_tiling_on_sc=False),
        out_type=jax.ShapeDtypeStruct((_ROWS, EMBED), jnp.float32),
        scratch_types=[
            pltpu.VMEM((_BPW, _NF32), jnp.int32),
            pltpu.VMEM((_CROWS, EMBED), jnp.float32),
            pltpu.VMEM((_CROWS, EMBED), jnp.float32),
            pltpu.VMEM((_CROWS, EMBED), jnp.float32),
            pltpu.SemaphoreType.DMA,
            pltpu.SemaphoreType.DMA,
            pltpu.SemaphoreType.DMA,
            pltpu.SemaphoreType.DMA,
            pltpu.SemaphoreType.DMA,
            pltpu.SemaphoreType.DMA,
        ],
    )
    def k(x_hbm, emb_hbm, out_hbm, idx_v, rows_a, rows_b, rows_c,
          gsem_a, gsem_b, gsem_c, osem_a, osem_b, osem_c):
        wid = lax.axis_index("s") * _NC + lax.axis_index("c")
        b0 = wid * _BPW
        bufs = (rows_a, rows_b, rows_c)
        gsems = (gsem_a, gsem_b, gsem_c)
        osems = (osem_a, osem_b, osem_c)
        pltpu.sync_copy(x_hbm.at[pl.ds(b0, _BPW), pl.ds(0, _NF32)], idx_v)

        def fire_gather(c):
            buf, sem = bufs[c % 3], gsems[c % 3]

            def body(i, carry):
                pltpu.async_copy(
                    emb_hbm.at[idx_v.at[c * _BPC + i]],
                    buf.at[pl.ds(i * _NF32, _NF32)], sem)
                return carry

            lax.fori_loop(0, _BPC, body, 0)

        def drain_gather(c):
            pltpu.make_async_copy(
                emb_hbm.at[pl.ds(0, _CROWS)], bufs[c % 3], gsems[c % 3]).wait()

        def fire_out(c):
            buf, sem = bufs[c % 3], osems[c % 3]
            base = (b0 + c * _BPC) * NFEAT

            def body(i, carry):
                pltpu.async_copy(
                    buf.at[pl.ds(i * _NF32, NFEAT)],
                    out_hbm.at[pl.ds(base + i * NFEAT, NFEAT)], sem)
                return carry

            lax.fori_loop(0, _BPC, body, 0)

        def drain_out(c):
            n = _BPC * NFEAT
            pltpu.make_async_copy(
                bufs[c % 3].at[pl.ds(0, n)],
                out_hbm.at[pl.ds(b0 * NFEAT, n)], osems[c % 3]).wait()

        fire_gather(0)
        fire_gather(1)
        for c in range(_NCHUNK):
            drain_gather(c)
            fire_out(c)
            if c >= 1:
                drain_out(c - 1)
            if c + 2 < _NCHUNK:
                fire_gather(c + 2)
        drain_out(_NCHUNK - 1)

    return k(x2d, emb)


_BB = 512


def _mlp_body(a_ref, w1_ref, b1_ref, w2_ref, b2_ref, o_ref):
    h = lax.dot_general(a_ref[...], w1_ref[...], (((1,), (1,)), ((), ())),
                        preferred_element_type=jnp.float32)
    h = jnp.maximum(h + b1_ref[...], 0.0)
    o = lax.dot_general(h, w2_ref[...], (((1,), (1,)), ((), ())),
                        preferred_element_type=jnp.float32)
    o_ref[...] = o + b2_ref[...]


def _mlp_tc(a, W1, b1, W2, b2):
    din = NFEAT * EMBED
    return pl.pallas_call(
        _mlp_body,
        grid=(BATCH // _BB,),
        in_specs=[
            pl.BlockSpec((_BB, din), lambda i: (i, 0)),
            pl.BlockSpec((HIDDEN, din), lambda i: (0, 0)),
            pl.BlockSpec((1, HIDDEN), lambda i: (0, 0)),
            pl.BlockSpec((NCLASS, HIDDEN), lambda i: (0, 0)),
            pl.BlockSpec((1, NCLASS), lambda i: (0, 0)),
        ],
        out_specs=pl.BlockSpec((_BB, NCLASS), lambda i: (i, 0)),
        out_shape=jax.ShapeDtypeStruct((BATCH, NCLASS), jnp.float32),
    )(a, W1, b1.reshape(1, HIDDEN), W2, b2.reshape(1, NCLASS))


def kernel(x, emb, W1, b1, W2, b2):
    xp = jnp.pad(x.astype(jnp.int32), ((0, 0), (0, 128 - NFEAT)))
    gathered = _gather_sc(xp, emb)
    a = gathered.reshape(BATCH, NFEAT * EMBED)
    return _mlp_tc(a, W1, b1, W2, b2)


# speedup vs baseline: 4.0787x; 4.0787x over previous
import functools

import jax
import jax.numpy as jnp
from jax import lax
from jax.experimental import pallas as pl
from jax.experimental.pallas import tpu as pltpu
from jax.experimental.pallas import tpu_sc as plsc

VOCAB = 100000
EMBED = 64
NFEAT = 26
HIDDEN = 1024
NCLASS = 2
BATCH = 4096

_NC = 2
_NS = 16
_NW = _NC * _NS

_ROWS = BATCH * NFEAT
_BPW = BATCH // _NW
_BPC = 32
_NCHUNK = _BPW // _BPC
_NF32 = 32
_CROWS = _BPC * NFEAT
_BROWS = _CROWS + (_NF32 - NFEAT)


def _pad_body(x_ref, o_ref):
    xs = x_ref[...]
    nxt = jnp.concatenate([xs[1:, :_NF32 - NFEAT], xs[:1, :_NF32 - NFEAT]],
                          axis=0)
    zero = jnp.zeros((BATCH, 128 - _NF32), jnp.int32)
    o_ref[...] = jnp.concatenate([xs, nxt, zero], axis=1)


def _pad_tc(x2d):
    return pl.pallas_call(
        _pad_body,
        out_shape=jax.ShapeDtypeStruct((BATCH, 128), jnp.int32),
    )(x2d)


def _gather_sc(xp, emb):
    mesh = plsc.VectorSubcoreMesh(core_axis_name="c", subcore_axis_name="s")

    @functools.partial(
        pl.kernel,
        mesh=mesh,
        compiler_params=pltpu.CompilerParams(use_tc_tiling_on_sc=False),
        out_type=jax.ShapeDtypeStruct((_ROWS, EMBED), jnp.float32),
        scratch_types=[
            pltpu.VMEM((_BPW, _NF32), jnp.int32),
            pltpu.VMEM((_BROWS, EMBED), jnp.float32),
            pltpu.VMEM((_BROWS, EMBED), jnp.float32),
            pltpu.SemaphoreType.DMA,
            pltpu.SemaphoreType.DMA,
        ],
    )
    def k(x_hbm, emb_hbm, out_hbm, idx_v, rows_a, rows_b, sem_a, sem_b):
        wid = lax.axis_index("s") * _NC + lax.axis_index("c")
        b0 = wid * _BPW
        bufs = (rows_a, rows_b)
        sems = (sem_a, sem_b)
        pltpu.sync_copy(x_hbm.at[pl.ds(b0, _BPW), pl.ds(0, _NF32)], idx_v)

        def fire(c):
            buf, sem = bufs[c % 2], sems[c % 2]

            def body(i, carry):
                pltpu.async_copy(
                    emb_hbm.at[idx_v.at[c * _BPC + i]],
                    buf.at[pl.ds(i * NFEAT, _NF32)], sem)
                return carry

            lax.fori_loop(0, _BPC, body, 0)

        def drain(c):
            total = _BPC * _NF32
            pltpu.make_async_copy(
                emb_hbm.at[pl.ds(0, _BROWS)], bufs[c % 2], sems[c % 2]).wait()
            pltpu.make_async_copy(
                emb_hbm.at[pl.ds(0, total - _BROWS)],
                bufs[c % 2].at[pl.ds(0, total - _BROWS)], sems[c % 2]).wait()

        fire(0)
        fire(1)
        for c in range(_NCHUNK):
            drain(c)
            pltpu.sync_copy(
                bufs[c % 2].at[pl.ds(0, _CROWS)],
                out_hbm.at[pl.ds((b0 + c * _BPC) * NFEAT, _CROWS)])
            if c + 2 < _NCHUNK:
                fire(c + 2)

    return k(xp, emb)


_BB = 512


def _mlp_body(a_ref, w1_ref, b1_ref, w2_ref, b2_ref, o_ref):
    h = lax.dot_general(a_ref[...], w1_ref[...], (((1,), (1,)), ((), ())),
                        preferred_element_type=jnp.float32)
    h = jnp.maximum(h + b1_ref[...], 0.0)
    o = lax.dot_general(h, w2_ref[...], (((1,), (1,)), ((), ())),
                        preferred_element_type=jnp.float32)
    o_ref[...] = o + b2_ref[...]


def _mlp_tc(a, W1, b1, W2, b2):
    din = NFEAT * EMBED
    return pl.pallas_call(
        _mlp_body,
        grid=(BATCH // _BB,),
        in_specs=[
            pl.BlockSpec((_BB, din), lambda i: (i, 0)),
            pl.BlockSpec((HIDDEN, din), lambda i: (0, 0)),
            pl.BlockSpec((1, HIDDEN), lambda i: (0, 0)),
            pl.BlockSpec((NCLASS, HIDDEN), lambda i: (0, 0)),
            pl.BlockSpec((1, NCLASS), lambda i: (0, 0)),
        ],
        out_specs=pl.BlockSpec((_BB, NCLASS), lambda i: (i, 0)),
        out_shape=jax.ShapeDtypeStruct((BATCH, NCLASS), jnp.float32),
    )(a, W1, b1.reshape(1, HIDDEN), W2, b2.reshape(1, NCLASS))


def kernel(x, emb, W1, b1, W2, b2):
    xp = _pad_tc(x.astype(jnp.int32))
    gathered = _gather_sc(xp, emb)
    a = gathered.reshape(BATCH, NFEAT * EMBED)
    return _mlp_tc(a, W1, b1, W2, b2)
